# Initial kernel scaffold; baseline (speedup 1.0000x reference)
import jax
import jax.numpy as jnp
from jax.experimental import pallas as pl


def kernel(x, params, edge_index, batch):
    raise NotImplementedError("write your pallas kernel here")


# speedup vs baseline: 2.0652x; 2.0652x over previous
import functools

import jax
import jax.numpy as jnp
from jax import lax
from jax.experimental import pallas as pl
from jax.experimental.pallas import tpu as pltpu
from jax.experimental.pallas import tpu_sc as plsc

_N = 10000
_E = 160000
_G = 64
_H = 256
_K = 128
_EPAD = 163840
_NM = 9592
_ZR = 600
_TB = 416
_R = 1000
_HI = lax.Precision.HIGHEST


def _seg_loop(gather_ref, acc, idx_s, idx_d, buf0, buf1, sem0, sem1, nchunks):
    dummy = gather_ref.at[pl.ds(0, _K)]

    pltpu.async_copy(gather_ref.at[idx_s.at[0]], buf0, sem0)

    def body(i, carry):
        j0 = 2 * i
        pltpu.async_copy(gather_ref.at[idx_s.at[j0 + 1]], buf1, sem1)
        pltpu.make_async_copy(dummy, buf0, sem0).wait()
        pltpu.sync_copy(buf0, acc.at[idx_d.at[j0]], add=True)

        @pl.when(j0 + 2 < nchunks)
        def _():
            pltpu.async_copy(gather_ref.at[idx_s.at[j0 + 2]], buf0, sem0)

        pltpu.make_async_copy(dummy, buf1, sem1).wait()
        pltpu.sync_copy(buf1, acc.at[idx_d.at[j0 + 1]], add=True)
        return carry

    lax.fori_loop(0, nchunks // 2, body, 0)


def _sc_body(nchunks, split32, gather, src_i, dst_i, zrow, out,
             idx_s, idx_d, buf0, buf1, acc, sem0, sem1):
    c = lax.axis_index("c")
    s = lax.axis_index("s")
    pltpu.sync_copy(zrow, acc.at[pl.ds(s * _ZR, _ZR)])
    if split32:
        wid = c * 16 + s
        gref = gather
    else:
        wid = s
        gref = gather.at[c]
    pltpu.sync_copy(src_i.at[wid], idx_s)
    pltpu.sync_copy(dst_i.at[wid], idx_d)

    def clamp_dst(j, carry):
        for k in range(8):
            v = idx_d[j, pl.ds(k * 16, 16)]
            idx_d[j, pl.ds(k * 16, 16)] = jnp.minimum(v, _NM)
        return carry

    lax.fori_loop(0, nchunks, clamp_dst, 0)
    plsc.subcore_barrier()

    _seg_loop(gref, acc, idx_s, idx_d, buf0, buf1, sem0, sem1, nchunks)
    plsc.subcore_barrier()

    @pl.when(s < 15)
    def _():
        pltpu.sync_copy(acc.at[pl.ds(s * _ZR, _ZR)],
                        out.at[c, pl.ds(s * _ZR, _ZR)])

    @pl.when(s == 15)
    def _():
        pltpu.sync_copy(acc.at[pl.ds(15 * _ZR, _NM - 15 * _ZR)],
                        out.at[c, pl.ds(15 * _ZR, _NM - 15 * _ZR)])

    plsc.subcore_barrier()

    pltpu.sync_copy(zrow.at[pl.ds(0, 32)], acc.at[pl.ds(s * 32, 32)])
    pltpu.sync_copy(dst_i.at[wid], idx_d)

    def tail_dst(j, carry):
        for k in range(8):
            v = idx_d[j, pl.ds(k * 16, 16)]
            idx_d[j, pl.ds(k * 16, 16)] = jnp.where(v >= _NM, v - _NM, _TB)
        return carry

    lax.fori_loop(0, nchunks, tail_dst, 0)
    plsc.subcore_barrier()
    _seg_loop(gref, acc, idx_s, idx_d, buf0, buf1, sem0, sem1, nchunks)
    plsc.subcore_barrier()

    @pl.when(s < 3)
    def _():
        pltpu.sync_copy(acc.at[pl.ds(s * 136, 136)],
                        out.at[c, pl.ds(_NM + s * 136, 136)])


@functools.cache
def _sc_segsum(nchunks, split32):
    return pl.kernel(
        functools.partial(_sc_body, nchunks, split32),
        mesh=plsc.VectorSubcoreMesh(core_axis_name="c", subcore_axis_name="s"),
        out_type=jax.ShapeDtypeStruct((2, _N, 128), jnp.float32),
        scratch_types=[
            pltpu.VMEM((nchunks, _K), jnp.int32),
            pltpu.VMEM((nchunks, _K), jnp.int32),
            pltpu.VMEM((_K, 128), jnp.float32),
            pltpu.VMEM((_K, 128), jnp.float32),
            pltpu.VMEM_SHARED((16 * _ZR, 128), jnp.float32),
            pltpu.SemaphoreType.DMA,
            pltpu.SemaphoreType.DMA,
        ],
    )


def _sc16(f0, src32, dst32, z128):
    return _sc_segsum(40, True)(f0, src32, dst32, z128)


def _sc128(p, src16, dst16, z128):
    return _sc_segsum(80, False)(p, src16, dst16, z128)


def _f0_body(x_ref, o_ref):
    xb = x_ref[...]
    r = xb.shape[0]
    li = lax.broadcasted_iota(jnp.int32, (r, 128), 1)
    v0 = xb[:, 0:1].astype(jnp.int32)
    v1 = xb[:, 1:2].astype(jnp.int32)
    oh0 = (li == v0).astype(jnp.float32)
    oh1 = ((li - 6) == v1).astype(jnp.float32)
    x2 = jnp.broadcast_to(xb[:, 2:3], (r, 128))
    x3 = jnp.broadcast_to(xb[:, 3:4], (r, 128))
    o_ref[...] = jnp.where(
        li < 6, oh0,
        jnp.where(li < 12, oh1,
                  jnp.where(li == 12, x2,
                            jnp.where(li == 13, x3,
                                      jnp.where(li == 14, 1.0, 0.0)))))


def _build_f0(x):
    return pl.pallas_call(
        _f0_body,
        grid=(_N // _R,),
        in_specs=[pl.BlockSpec((_R, 4), lambda i: (i, 0))],
        out_specs=pl.BlockSpec((_R, 128), lambda i: (i, 0)),
        out_shape=jax.ShapeDtypeStruct((_N, 128), jnp.float32),
    )(x)


def _write_sums(s_ref, h2):
    row = jnp.concatenate(
        [jnp.sum(h2, axis=0, keepdims=True),
         jnp.sum(h2 * h2, axis=0, keepdims=True)], axis=0)

    @pl.when(pl.program_id(0) == 0)
    def _():
        s_ref[...] = row

    @pl.when(pl.program_id(0) != 0)
    def _():
        s_ref[...] = s_ref[...] + row


def _conv0_body(f0_ref, g0_ref, emb_ref, w1t_ref, b1_ref, w2t_ref, b2_ref,
                eps_ref, p_ref, s_ref):
    f0b = f0_ref[:, 0:16]
    g0 = g0_ref[0, :, 0:16] + g0_ref[1, :, 0:16]
    z = (1.0 + eps_ref[0, 0]) * f0b + g0
    emb = emb_ref[...]
    z0a = jnp.dot(z[:, 0:6], emb, precision=_HI)
    z0b = jnp.dot(z[:, 6:12], emb, precision=_HI)
    pre = (jnp.dot(z0a, w1t_ref[0:128, :])
           + jnp.dot(z0b, w1t_ref[128:256, :])
           + jnp.dot(z[:, 12:14], w1t_ref[256:258, :])
           + b1_ref[...])
    h1 = jnp.maximum(pre, 0.0)
    h2 = jnp.maximum(jnp.dot(h1, w2t_ref[...]) + b2_ref[...], 0.0)
    p_ref[0] = h2[:, 0:128]
    p_ref[1] = h2[:, 128:256]
    _write_sums(s_ref, h2)


def _conv0(f0, g0p, emb, w1t, b1, w2t, b2, eps):
    return pl.pallas_call(
        _conv0_body,
        grid=(_N // _R,),
        in_specs=[
            pl.BlockSpec((_R, 128), lambda i: (i, 0)),
            pl.BlockSpec((2, _R, 128), lambda i: (0, i, 0)),
            pl.BlockSpec((6, 128), lambda i: (0, 0)),
            pl.BlockSpec((258, 256), lambda i: (0, 0)),
            pl.BlockSpec((1, 256), lambda i: (0, 0)),
            pl.BlockSpec((256, 256), lambda i: (0, 0)),
            pl.BlockSpec((1, 256), lambda i: (0, 0)),
            pl.BlockSpec((1, 1), lambda i: (0, 0)),
        ],
        out_specs=[
            pl.BlockSpec((2, _R, 128), lambda i: (0, i, 0)),
            pl.BlockSpec((2, 256), lambda i: (0, 0)),
        ],
        out_shape=[
            jax.ShapeDtypeStruct((2, _N, 128), jnp.float32),
            jax.ShapeDtypeStruct((2, 256), jnp.float32),
        ],
    )(f0, g0p, emb, w1t, b1, w2t, b2, eps)


def _bn_scale(sums, gamma, beta):
    mean = sums[0:1, :] * (1.0 / _N)
    var = sums[1:2, :] * (1.0 / _N) - mean * mean
    s = gamma * lax.rsqrt(var + 1e-5)
    t = beta - mean * s
    return s, t


def _convl_body(pp_ref, ag_ref, g0_ref, sp_ref, gam_ref, bet_ref, eps_ref,
                w1t_ref, b1_ref, w2t_ref, b2_ref, p_ref, s_ref):
    s_bn, t_bn = _bn_scale(sp_ref[...], gam_ref[...], bet_ref[...])
    pprev = jnp.concatenate([pp_ref[0], pp_ref[1]], axis=1)
    agg = jnp.concatenate([ag_ref[0], ag_ref[1]], axis=1)
    indeg = g0_ref[0, :, 14:15] + g0_ref[1, :, 14:15]
    e1 = 1.0 + eps_ref[0, 0]
    z = s_bn * (e1 * pprev + agg) + t_bn * (e1 + indeg)
    h1 = jnp.maximum(jnp.dot(z, w1t_ref[...]) + b1_ref[...], 0.0)
    h2 = jnp.maximum(jnp.dot(h1, w2t_ref[...]) + b2_ref[...], 0.0)
    p_ref[0] = h2[:, 0:128]
    p_ref[1] = h2[:, 128:256]
    _write_sums(s_ref, h2)


def _convl(pp, ag, g0p, sums_p, gam, bet, eps, w1t, b1, w2t, b2):
    return pl.pallas_call(
        _convl_body,
        grid=(_N // _R,),
        in_specs=[
            pl.BlockSpec((2, _R, 128), lambda i: (0, i, 0)),
            pl.BlockSpec((2, _R, 128), lambda i: (0, i, 0)),
            pl.BlockSpec((2, _R, 128), lambda i: (0, i, 0)),
            pl.BlockSpec((2, 256), lambda i: (0, 0)),
            pl.BlockSpec((1, 256), lambda i: (0, 0)),
            pl.BlockSpec((1, 256), lambda i: (0, 0)),
            pl.BlockSpec((1, 1), lambda i: (0, 0)),
            pl.BlockSpec((256, 256), lambda i: (0, 0)),
            pl.BlockSpec((1, 256), lambda i: (0, 0)),
            pl.BlockSpec((256, 256), lambda i: (0, 0)),
            pl.BlockSpec((1, 256), lambda i: (0, 0)),
        ],
        out_specs=[
            pl.BlockSpec((2, _R, 128), lambda i: (0, i, 0)),
            pl.BlockSpec((2, 256), lambda i: (0, 0)),
        ],
        out_shape=[
            jax.ShapeDtypeStruct((2, _N, 128), jnp.float32),
            jax.ShapeDtypeStruct((2, 256), jnp.float32),
        ],
    )(pp, ag, g0p, sums_p, gam, bet, eps, w1t, b1, w2t, b2)


def _head_body(p0_ref, p1_ref, p2_ref, s0_ref, s1_ref, s2_ref,
               gam_ref, bet_ref, bc_ref, br_ref,
               wiht_ref, bih_ref, whht_ref, bhh_ref,
               f1t_ref, f1b_ref, f2t_ref, f2b_ref, f3t_ref, f3b_ref,
               o_ref, x_scr):
    nch = _N // _R
    p_refs = (p0_ref, p1_ref, p2_ref)
    s_refs = (s0_ref, s1_ref, s2_ref)

    def build_x(k, carry):
        xc = jnp.full((_R, _H), -jnp.inf, jnp.float32)
        for l in range(3):
            s_bn, t_bn = _bn_scale(s_refs[l][...], gam_ref[l:l + 1, :],
                                   bet_ref[l:l + 1, :])
            pb = jnp.concatenate(
                [p_refs[l][0, pl.ds(k * _R, _R), :],
                 p_refs[l][1, pl.ds(k * _R, _R), :]], axis=1)
            xc = jnp.maximum(xc, s_bn * pb + t_bn)
        x_scr[pl.ds(k * _R, _R), :] = xc
        return carry

    lax.fori_loop(0, nch, build_x, 0)

    h = jnp.zeros((_G, _H), jnp.float32)
    cell = jnp.zeros((_G, _H), jnp.float32)
    qs = jnp.zeros((_G, 2 * _H), jnp.float32)
    for _step in range(3):
        gates = (jnp.dot(qs, wiht_ref[...]) + bih_ref[...]
                 + jnp.dot(h, whht_ref[...]) + bhh_ref[...])
        gi = jax.nn.sigmoid(gates[:, 0:256])
        gf = jax.nn.sigmoid(gates[:, 256:512])
        gg = jnp.tanh(gates[:, 512:768])
        go = jax.nn.sigmoid(gates[:, 768:1024])
        cell = gf * cell + gi * gg
        h = go * jnp.tanh(cell)

        def pass1(k, emax):
            xb = x_scr[pl.ds(k * _R, _R), :]
            bc = bc_ref[pl.ds(k * _R, _R), :].astype(jnp.int32)
            mk = bc == lax.broadcasted_iota(jnp.int32, (_R, _G), 1)
            qb = jnp.dot(mk.astype(jnp.float32), h, precision=_HI)
            e = jnp.sum(xb * qb, axis=1, keepdims=True)
            em = jnp.where(mk, e, -1e30)
            return jnp.maximum(emax, jnp.max(em, axis=0, keepdims=True))

        emax = lax.fori_loop(0, nch, pass1,
                             jnp.full((1, _G), -1e30, jnp.float32))

        def pass2(k, carry):
            den, run = carry
            xb = x_scr[pl.ds(k * _R, _R), :]
            bc = bc_ref[pl.ds(k * _R, _R), :].astype(jnp.int32)
            mk = bc == lax.broadcasted_iota(jnp.int32, (_R, _G), 1)
            qb = jnp.dot(mk.astype(jnp.float32), h, precision=_HI)
            e = jnp.sum(xb * qb, axis=1, keepdims=True)
            emaxb = jnp.max(jnp.where(mk, emax, -1e30), axis=1, keepdims=True)
            ex = jnp.exp(e - emaxb)
            br = br_ref[k].astype(jnp.int32)
            mkt = (br == lax.broadcasted_iota(jnp.int32, (_G, _R), 0)
                   ).astype(jnp.float32)
            den = den + jnp.dot(mkt, ex, precision=_HI)
            run = run + jnp.dot(mkt, ex * xb, precision=_HI)
            return den, run

        den, run = lax.fori_loop(
            0, nch, pass2,
            (jnp.zeros((_G, 1), jnp.float32), jnp.zeros((_G, _H), jnp.float32)))
        r = run / jnp.maximum(den, 1e-30)
        qs = jnp.concatenate([h, r], axis=1)

    h4 = jnp.maximum(jnp.dot(qs, f1t_ref[...]) + f1b_ref[...], 0.0)
    h5 = jnp.maximum(jnp.dot(h4, f2t_ref[...]) + f2b_ref[...], 0.0)
    o_ref[...] = jnp.dot(h5, f3t_ref[...]) + f3b_ref[...]


def _head(p0, p1, p2, s0, s1, s2, gam, bet, bc, br3, wiht, bih, whht, bhh,
          f1t, f1b, f2t, f2b, f3t, f3b):
    full = lambda shape: pl.BlockSpec(shape, lambda: tuple(0 for _ in shape))
    return pl.pallas_call(
        _head_body,
        grid=(),
        in_specs=[
            full((2, _N, 128)), full((2, _N, 128)), full((2, _N, 128)),
            full((2, 256)), full((2, 256)), full((2, 256)),
            full((3, 256)), full((3, 256)),
            full((_N, 1)), full((_N // _R, 1, _R)),
            full((2 * _H, 4 * _H)), full((1, 4 * _H)),
            full((_H, 4 * _H)), full((1, 4 * _H)),
            full((2 * _H, _H)), full((1, _H)),
            full((_H, _H // 2)), full((1, _H // 2)),
            full((_H // 2, 2)), full((1, 2)),
        ],
        out_specs=full((_G, 2)),
        out_shape=jax.ShapeDtypeStruct((_G, 2), jnp.float32),
        scratch_shapes=[pltpu.VMEM((_N, _H), jnp.float32)],
    )(p0, p1, p2, s0, s1, s2, gam, bet, bc, br3, wiht, bih, whht, bhh,
      f1t, f1b, f2t, f2b, f3t, f3b)


def kernel(x, params, edge_index, batch):
    src = edge_index[0].astype(jnp.int32)
    dst = edge_index[1].astype(jnp.int32)
    pad = _EPAD - _E
    srcp = jnp.concatenate([src, jnp.zeros((pad,), jnp.int32)])
    dstp = jnp.concatenate([dst, jnp.full((pad,), _NM + _TB, jnp.int32)])
    src32 = srcp.reshape(32, 40, _K)
    dst32 = dstp.reshape(32, 40, _K)
    src16 = srcp.reshape(16, 80, _K)
    dst16 = dstp.reshape(16, 80, _K)
    z128 = jnp.zeros((_ZR, 128), jnp.float32)

    convs = params["convs"]
    w1t = [c["W1"].T for c in convs]
    w2t = [c["W2"].T for c in convs]
    b1 = [c["b1"].reshape(1, _H) for c in convs]
    b2 = [c["b2"].reshape(1, _H) for c in convs]
    eps = [c["eps"].reshape(1, 1) for c in convs]
    gam = jnp.stack([c["gamma"] for c in convs])
    bet = jnp.stack([c["beta"] for c in convs])

    f0 = _build_f0(x)
    g0p = _sc16(f0, src32, dst32, z128)
    p0, s0 = _conv0(f0, g0p, params["emb"], w1t[0], b1[0], w2t[0], b2[0],
                    eps[0])
    agg0 = _sc128(p0, src16, dst16, z128)
    p1, s1 = _convl(p0, agg0, g0p, s0, gam[0:1], bet[0:1], eps[1],
                    w1t[1], b1[1], w2t[1], b2[1])
    agg1 = _sc128(p1, src16, dst16, z128)
    p2, s2 = _convl(p1, agg1, g0p, s1, gam[1:2], bet[1:2], eps[2],
                    w1t[2], b1[2], w2t[2], b2[2])

    bc = batch.astype(jnp.float32).reshape(_N, 1)
    br3 = batch.astype(jnp.float32).reshape(_N // _R, 1, _R)
    logits = _head(
        p0, p1, p2, s0, s1, s2, gam, bet, bc, br3,
        params["Wih"].T, params["bih"].reshape(1, 4 * _H),
        params["Whh"].T, params["bhh"].reshape(1, 4 * _H),
        params["fc1W"].T, params["fc1b"].reshape(1, _H),
        params["fc2W"].T, params["fc2b"].reshape(1, _H // 2),
        params["fc3W"].T, params["fc3b"].reshape(1, 2))
    return logits


# speedup vs baseline: 2.1959x; 1.0633x over previous
import functools

import jax
import jax.numpy as jnp
from jax import lax
from jax.experimental import pallas as pl
from jax.experimental.pallas import tpu as pltpu
from jax.experimental.pallas import tpu_sc as plsc

_N = 10000
_E = 160000
_G = 64
_H = 256
_K = 128
_EPAD = 163840
_NM = 9592
_ZR = 600
_TB = 416
_R = 1000
_HI = lax.Precision.HIGHEST


def _seg_loop(gather_ref, acc, idx_s, idx_d, buf0, buf1, sem0, sem1, nchunks):
    dummy = gather_ref.at[pl.ds(0, _K)]

    pltpu.async_copy(gather_ref.at[idx_s.at[0]], buf0, sem0)

    def body(i, carry):
        j0 = 2 * i
        pltpu.async_copy(gather_ref.at[idx_s.at[j0 + 1]], buf1, sem1)
        pltpu.make_async_copy(dummy, buf0, sem0).wait()
        pltpu.sync_copy(buf0, acc.at[idx_d.at[j0]], add=True)

        @pl.when(j0 + 2 < nchunks)
        def _():
            pltpu.async_copy(gather_ref.at[idx_s.at[j0 + 2]], buf0, sem0)

        pltpu.make_async_copy(dummy, buf1, sem1).wait()
        pltpu.sync_copy(buf1, acc.at[idx_d.at[j0 + 1]], add=True)
        return carry

    lax.fori_loop(0, nchunks // 2, body, 0)


def _sc_body(nchunks, split32, gather, src_i, dst_i, zrow, out,
             idx_s, idx_d, buf0, buf1, acc, sem0, sem1):
    c = lax.axis_index("c")
    s = lax.axis_index("s")
    pltpu.sync_copy(zrow, acc.at[pl.ds(s * _ZR, _ZR)])
    if split32:
        wid = c * 16 + s
        gref = gather
    else:
        wid = s
        gref = gather.at[c]
    pltpu.sync_copy(src_i.at[wid], idx_s)
    pltpu.sync_copy(dst_i.at[wid], idx_d)

    iota16 = lax.broadcasted_iota(jnp.int32, (16,), 0)

    def clamp_dst(j, carry):
        for k in range(8):
            v = idx_d[j, pl.ds(k * 16, 16)]
            idx_d[j, pl.ds(k * 16, 16)] = jnp.where(
                v >= _NM, _NM + (iota16 & 7), v)
        return carry

    lax.fori_loop(0, nchunks, clamp_dst, 0)
    plsc.subcore_barrier()

    _seg_loop(gref, acc, idx_s, idx_d, buf0, buf1, sem0, sem1, nchunks)
    plsc.subcore_barrier()

    @pl.when(s < 15)
    def _():
        pltpu.sync_copy(acc.at[pl.ds(s * _ZR, _ZR)],
                        out.at[c, pl.ds(s * _ZR, _ZR)])

    @pl.when(s == 15)
    def _():
        pltpu.sync_copy(acc.at[pl.ds(15 * _ZR, _NM - 15 * _ZR)],
                        out.at[c, pl.ds(15 * _ZR, _NM - 15 * _ZR)])

    plsc.subcore_barrier()

    pltpu.sync_copy(zrow.at[pl.ds(0, 32)], acc.at[pl.ds(s * 32, 32)])
    pltpu.sync_copy(dst_i.at[wid], idx_d)

    def tail_dst(j, carry):
        for k in range(8):
            v = idx_d[j, pl.ds(k * 16, 16)]
            trash = _TB + ((iota16 + (j * 8 + k) * 16) & 63)
            idx_d[j, pl.ds(k * 16, 16)] = jnp.where(v >= _NM, v - _NM, trash)
        return carry

    lax.fori_loop(0, nchunks, tail_dst, 0)
    plsc.subcore_barrier()
    _seg_loop(gref, acc, idx_s, idx_d, buf0, buf1, sem0, sem1, nchunks)
    plsc.subcore_barrier()

    @pl.when(s < 3)
    def _():
        pltpu.sync_copy(acc.at[pl.ds(s * 136, 136)],
                        out.at[c, pl.ds(_NM + s * 136, 136)])


@functools.cache
def _sc_segsum(nchunks, split32):
    return pl.kernel(
        functools.partial(_sc_body, nchunks, split32),
        mesh=plsc.VectorSubcoreMesh(core_axis_name="c", subcore_axis_name="s"),
        out_type=jax.ShapeDtypeStruct((2, _N, 128), jnp.float32),
        scratch_types=[
            pltpu.VMEM((nchunks, _K), jnp.int32),
            pltpu.VMEM((nchunks, _K), jnp.int32),
            pltpu.VMEM((_K, 128), jnp.float32),
            pltpu.VMEM((_K, 128), jnp.float32),
            pltpu.VMEM_SHARED((16 * _ZR, 128), jnp.float32),
            pltpu.SemaphoreType.DMA,
            pltpu.SemaphoreType.DMA,
        ],
    )


def _sc16(f0, src32, dst32, z128):
    return _sc_segsum(40, True)(f0, src32, dst32, z128)


def _sc128(p, src16, dst16, z128):
    return _sc_segsum(80, False)(p, src16, dst16, z128)


def _f0_body(x_ref, o_ref):
    xb = x_ref[...]
    r = xb.shape[0]
    li = lax.broadcasted_iota(jnp.int32, (r, 128), 1)
    v0 = xb[:, 0:1].astype(jnp.int32)
    v1 = xb[:, 1:2].astype(jnp.int32)
    oh0 = (li == v0).astype(jnp.float32)
    oh1 = ((li - 6) == v1).astype(jnp.float32)
    x2 = jnp.broadcast_to(xb[:, 2:3], (r, 128))
    x3 = jnp.broadcast_to(xb[:, 3:4], (r, 128))
    o_ref[...] = jnp.where(
        li < 6, oh0,
        jnp.where(li < 12, oh1,
                  jnp.where(li == 12, x2,
                            jnp.where(li == 13, x3,
                                      jnp.where(li == 14, 1.0, 0.0)))))


def _build_f0(x):
    return pl.pallas_call(
        _f0_body,
        grid=(_N // _R,),
        in_specs=[pl.BlockSpec((_R, 4), lambda i: (i, 0))],
        out_specs=pl.BlockSpec((_R, 128), lambda i: (i, 0)),
        out_shape=jax.ShapeDtypeStruct((_N, 128), jnp.float32),
    )(x)


def _write_sums(s_ref, h2):
    row = jnp.concatenate(
        [jnp.sum(h2, axis=0, keepdims=True),
         jnp.sum(h2 * h2, axis=0, keepdims=True)], axis=0)

    @pl.when(pl.program_id(0) == 0)
    def _():
        s_ref[...] = row

    @pl.when(pl.program_id(0) != 0)
    def _():
        s_ref[...] = s_ref[...] + row


def _conv0_body(f0_ref, g0_ref, emb_ref, w1t_ref, b1_ref, w2t_ref, b2_ref,
                eps_ref, p_ref, s_ref):
    f0b = f0_ref[:, 0:16]
    g0 = g0_ref[0, :, 0:16] + g0_ref[1, :, 0:16]
    z = (1.0 + eps_ref[0, 0]) * f0b + g0
    emb = emb_ref[...]
    z0a = jnp.dot(z[:, 0:6], emb, precision=_HI)
    z0b = jnp.dot(z[:, 6:12], emb, precision=_HI)
    pre = (jnp.dot(z0a, w1t_ref[0:128, :])
           + jnp.dot(z0b, w1t_ref[128:256, :])
           + jnp.dot(z[:, 12:14], w1t_ref[256:258, :])
           + b1_ref[...])
    h1 = jnp.maximum(pre, 0.0)
    h2 = jnp.maximum(jnp.dot(h1, w2t_ref[...]) + b2_ref[...], 0.0)
    p_ref[0] = h2[:, 0:128]
    p_ref[1] = h2[:, 128:256]
    _write_sums(s_ref, h2)


def _conv0(f0, g0p, emb, w1t, b1, w2t, b2, eps):
    return pl.pallas_call(
        _conv0_body,
        grid=(_N // _R,),
        in_specs=[
            pl.BlockSpec((_R, 128), lambda i: (i, 0)),
            pl.BlockSpec((2, _R, 128), lambda i: (0, i, 0)),
            pl.BlockSpec((6, 128), lambda i: (0, 0)),
            pl.BlockSpec((258, 256), lambda i: (0, 0)),
            pl.BlockSpec((1, 256), lambda i: (0, 0)),
            pl.BlockSpec((256, 256), lambda i: (0, 0)),
            pl.BlockSpec((1, 256), lambda i: (0, 0)),
            pl.BlockSpec((1, 1), lambda i: (0, 0)),
        ],
        out_specs=[
            pl.BlockSpec((2, _R, 128), lambda i: (0, i, 0)),
            pl.BlockSpec((2, 256), lambda i: (0, 0)),
        ],
        out_shape=[
            jax.ShapeDtypeStruct((2, _N, 128), jnp.float32),
            jax.ShapeDtypeStruct((2, 256), jnp.float32),
        ],
    )(f0, g0p, emb, w1t, b1, w2t, b2, eps)


def _bn_scale(sums, gamma, beta):
    mean = sums[0:1, :] * (1.0 / _N)
    var = sums[1:2, :] * (1.0 / _N) - mean * mean
    s = gamma * lax.rsqrt(var + 1e-5)
    t = beta - mean * s
    return s, t


def _convl_body(pp_ref, ag_ref, g0_ref, sp_ref, gam_ref, bet_ref, eps_ref,
                w1t_ref, b1_ref, w2t_ref, b2_ref, p_ref, s_ref):
    s_bn, t_bn = _bn_scale(sp_ref[...], gam_ref[...], bet_ref[...])
    pprev = jnp.concatenate([pp_ref[0], pp_ref[1]], axis=1)
    agg = jnp.concatenate([ag_ref[0], ag_ref[1]], axis=1)
    indeg = g0_ref[0, :, 14:15] + g0_ref[1, :, 14:15]
    e1 = 1.0 + eps_ref[0, 0]
    z = s_bn * (e1 * pprev + agg) + t_bn * (e1 + indeg)
    h1 = jnp.maximum(jnp.dot(z, w1t_ref[...]) + b1_ref[...], 0.0)
    h2 = jnp.maximum(jnp.dot(h1, w2t_ref[...]) + b2_ref[...], 0.0)
    p_ref[0] = h2[:, 0:128]
    p_ref[1] = h2[:, 128:256]
    _write_sums(s_ref, h2)


def _convl(pp, ag, g0p, sums_p, gam, bet, eps, w1t, b1, w2t, b2):
    return pl.pallas_call(
        _convl_body,
        grid=(_N // _R,),
        in_specs=[
            pl.BlockSpec((2, _R, 128), lambda i: (0, i, 0)),
            pl.BlockSpec((2, _R, 128), lambda i: (0, i, 0)),
            pl.BlockSpec((2, _R, 128), lambda i: (0, i, 0)),
            pl.BlockSpec((2, 256), lambda i: (0, 0)),
            pl.BlockSpec((1, 256), lambda i: (0, 0)),
            pl.BlockSpec((1, 256), lambda i: (0, 0)),
            pl.BlockSpec((1, 1), lambda i: (0, 0)),
            pl.BlockSpec((256, 256), lambda i: (0, 0)),
            pl.BlockSpec((1, 256), lambda i: (0, 0)),
            pl.BlockSpec((256, 256), lambda i: (0, 0)),
            pl.BlockSpec((1, 256), lambda i: (0, 0)),
        ],
        out_specs=[
            pl.BlockSpec((2, _R, 128), lambda i: (0, i, 0)),
            pl.BlockSpec((2, 256), lambda i: (0, 0)),
        ],
        out_shape=[
            jax.ShapeDtypeStruct((2, _N, 128), jnp.float32),
            jax.ShapeDtypeStruct((2, 256), jnp.float32),
        ],
    )(pp, ag, g0p, sums_p, gam, bet, eps, w1t, b1, w2t, b2)


def _head_body(p0_ref, p1_ref, p2_ref, s0_ref, s1_ref, s2_ref,
               gam_ref, bet_ref, bc_ref, br_ref,
               wiht_ref, bih_ref, whht_ref, bhh_ref,
               f1t_ref, f1b_ref, f2t_ref, f2b_ref, f3t_ref, f3b_ref,
               o_ref, x_scr):
    nch = _N // _R
    p_refs = (p0_ref, p1_ref, p2_ref)
    s_refs = (s0_ref, s1_ref, s2_ref)

    def build_x(k, carry):
        xc = jnp.full((_R, _H), -jnp.inf, jnp.float32)
        for l in range(3):
            s_bn, t_bn = _bn_scale(s_refs[l][...], gam_ref[l:l + 1, :],
                                   bet_ref[l:l + 1, :])
            pb = jnp.concatenate(
                [p_refs[l][0, pl.ds(k * _R, _R), :],
                 p_refs[l][1, pl.ds(k * _R, _R), :]], axis=1)
            xc = jnp.maximum(xc, s_bn * pb + t_bn)
        x_scr[pl.ds(k * _R, _R), :] = xc
        return carry

    lax.fori_loop(0, nch, build_x, 0)

    h = jnp.zeros((_G, _H), jnp.float32)
    cell = jnp.zeros((_G, _H), jnp.float32)
    qs = jnp.zeros((_G, 2 * _H), jnp.float32)
    for _step in range(3):
        gates = (jnp.dot(qs, wiht_ref[...]) + bih_ref[...]
                 + jnp.dot(h, whht_ref[...]) + bhh_ref[...])
        gi = jax.nn.sigmoid(gates[:, 0:256])
        gf = jax.nn.sigmoid(gates[:, 256:512])
        gg = jnp.tanh(gates[:, 512:768])
        go = jax.nn.sigmoid(gates[:, 768:1024])
        cell = gf * cell + gi * gg
        h = go * jnp.tanh(cell)

        def pass1(k, emax):
            xb = x_scr[pl.ds(k * _R, _R), :]
            bc = bc_ref[pl.ds(k * _R, _R), :].astype(jnp.int32)
            mk = bc == lax.broadcasted_iota(jnp.int32, (_R, _G), 1)
            qb = jnp.dot(mk.astype(jnp.float32), h, precision=_HI)
            e = jnp.sum(xb * qb, axis=1, keepdims=True)
            em = jnp.where(mk, e, -1e30)
            return jnp.maximum(emax, jnp.max(em, axis=0, keepdims=True))

        emax = lax.fori_loop(0, nch, pass1,
                             jnp.full((1, _G), -1e30, jnp.float32))

        def pass2(k, carry):
            den, run = carry
            xb = x_scr[pl.ds(k * _R, _R), :]
            bc = bc_ref[pl.ds(k * _R, _R), :].astype(jnp.int32)
            mk = bc == lax.broadcasted_iota(jnp.int32, (_R, _G), 1)
            qb = jnp.dot(mk.astype(jnp.float32), h, precision=_HI)
            e = jnp.sum(xb * qb, axis=1, keepdims=True)
            emaxb = jnp.max(jnp.where(mk, emax, -1e30), axis=1, keepdims=True)
            ex = jnp.exp(e - emaxb)
            br = br_ref[k].astype(jnp.int32)
            mkt = (br == lax.broadcasted_iota(jnp.int32, (_G, _R), 0)
                   ).astype(jnp.float32)
            den = den + jnp.dot(mkt, ex, precision=_HI)
            run = run + jnp.dot(mkt, ex * xb, precision=_HI)
            return den, run

        den, run = lax.fori_loop(
            0, nch, pass2,
            (jnp.zeros((_G, 1), jnp.float32), jnp.zeros((_G, _H), jnp.float32)))
        r = run / jnp.maximum(den, 1e-30)
        qs = jnp.concatenate([h, r], axis=1)

    h4 = jnp.maximum(jnp.dot(qs, f1t_ref[...]) + f1b_ref[...], 0.0)
    h5 = jnp.maximum(jnp.dot(h4, f2t_ref[...]) + f2b_ref[...], 0.0)
    o_ref[...] = jnp.dot(h5, f3t_ref[...]) + f3b_ref[...]


def _head(p0, p1, p2, s0, s1, s2, gam, bet, bc, br3, wiht, bih, whht, bhh,
          f1t, f1b, f2t, f2b, f3t, f3b):
    full = lambda shape: pl.BlockSpec(shape, lambda: tuple(0 for _ in shape))
    return pl.pallas_call(
        _head_body,
        grid=(),
        in_specs=[
            full((2, _N, 128)), full((2, _N, 128)), full((2, _N, 128)),
            full((2, 256)), full((2, 256)), full((2, 256)),
            full((3, 256)), full((3, 256)),
            full((_N, 1)), full((_N // _R, 1, _R)),
            full((2 * _H, 4 * _H)), full((1, 4 * _H)),
            full((_H, 4 * _H)), full((1, 4 * _H)),
            full((2 * _H, _H)), full((1, _H)),
            full((_H, _H // 2)), full((1, _H // 2)),
            full((_H // 2, 2)), full((1, 2)),
        ],
        out_specs=full((_G, 2)),
        out_shape=jax.ShapeDtypeStruct((_G, 2), jnp.float32),
        scratch_shapes=[pltpu.VMEM((_N, _H), jnp.float32)],
    )(p0, p1, p2, s0, s1, s2, gam, bet, bc, br3, wiht, bih, whht, bhh,
      f1t, f1b, f2t, f2b, f3t, f3b)


def kernel(x, params, edge_index, batch):
    src = edge_index[0].astype(jnp.int32)
    dst = edge_index[1].astype(jnp.int32)
    pad = _EPAD - _E
    srcp = jnp.concatenate([src, jnp.zeros((pad,), jnp.int32)])
    dstp = jnp.concatenate([dst, jnp.full((pad,), _NM + _TB, jnp.int32)])
    src32 = srcp.reshape(32, 40, _K)
    dst32 = dstp.reshape(32, 40, _K)
    src16 = srcp.reshape(16, 80, _K)
    dst16 = dstp.reshape(16, 80, _K)
    z128 = jnp.zeros((_ZR, 128), jnp.float32)

    convs = params["convs"]
    w1t = [c["W1"].T for c in convs]
    w2t = [c["W2"].T for c in convs]
    b1 = [c["b1"].reshape(1, _H) for c in convs]
    b2 = [c["b2"].reshape(1, _H) for c in convs]
    eps = [c["eps"].reshape(1, 1) for c in convs]
    gam = jnp.stack([c["gamma"] for c in convs])
    bet = jnp.stack([c["beta"] for c in convs])

    f0 = _build_f0(x)
    g0p = _sc16(f0, src32, dst32, z128)
    p0, s0 = _conv0(f0, g0p, params["emb"], w1t[0], b1[0], w2t[0], b2[0],
                    eps[0])
    agg0 = _sc128(p0, src16, dst16, z128)
    p1, s1 = _convl(p0, agg0, g0p, s0, gam[0:1], bet[0:1], eps[1],
                    w1t[1], b1[1], w2t[1], b2[1])
    agg1 = _sc128(p1, src16, dst16, z128)
    p2, s2 = _convl(p1, agg1, g0p, s1, gam[1:2], bet[1:2], eps[2],
                    w1t[2], b1[2], w2t[2], b2[2])

    bc = batch.astype(jnp.float32).reshape(_N, 1)
    br3 = batch.astype(jnp.float32).reshape(_N // _R, 1, _R)
    logits = _head(
        p0, p1, p2, s0, s1, s2, gam, bet, bc, br3,
        params["Wih"].T, params["bih"].reshape(1, 4 * _H),
        params["Whh"].T, params["bhh"].reshape(1, 4 * _H),
        params["fc1W"].T, params["fc1b"].reshape(1, _H),
        params["fc2W"].T, params["fc2b"].reshape(1, _H // 2),
        params["fc3W"].T, params["fc3b"].reshape(1, 2))
    return logits


# speedup vs baseline: 3.7889x; 1.7255x over previous
import functools

import jax
import jax.numpy as jnp
from jax import lax
from jax.experimental import pallas as pl
from jax.experimental.pallas import tpu as pltpu
from jax.experimental.pallas import tpu_sc as plsc

_N = 10000
_E = 160000
_G = 64
_H = 256
_K = 128
_EPAD = 163840
_AR = 10016
_ZR = 632
_R = 1000
_HI = lax.Precision.HIGHEST


def _sc_body(nchunks, split32, gather, src_i, dst_i, zrow, out,
             sidx, didx, buf0, buf1, acc, sem0, sem1, semi):
    c = lax.axis_index("c")
    s = lax.axis_index("s")

    @pl.when(s < 15)
    def _():
        pltpu.sync_copy(zrow, acc.at[pl.ds(s * _ZR, _ZR)])

    @pl.when(s == 15)
    def _():
        pltpu.sync_copy(zrow.at[pl.ds(0, _AR - 15 * _ZR)],
                        acc.at[pl.ds(15 * _ZR, _AR - 15 * _ZR)])

    if split32:
        wid = c * 16 + s
        gref = gather
    else:
        wid = s
        gref = gather.at[c]
    srow = src_i.at[wid]
    drow = dst_i.at[wid]
    plsc.subcore_barrier()

    ng = nchunks // 8
    pltpu.sync_copy(srow.at[pl.ds(0, 8)], sidx.at[0])
    pltpu.sync_copy(drow.at[pl.ds(0, 8)], didx.at[0])
    pltpu.async_copy(gref.at[sidx.at[0, 0]], buf0, sem0)
    gdummy = gref.at[pl.ds(0, _K)]
    idummy = srow.at[pl.ds(0, 8)]

    def group(g, carry):
        par = g % 2
        nxt = (g + 1) % 2

        @pl.when(g + 1 < ng)
        def _():
            pltpu.async_copy(srow.at[pl.ds((g + 1) * 8, 8)], sidx.at[nxt],
                             semi)
            pltpu.async_copy(drow.at[pl.ds((g + 1) * 8, 8)], didx.at[nxt],
                             semi)

        for b in range(8):
            buf, sem = (buf0, sem0) if b % 2 == 0 else (buf1, sem1)
            obuf, osem = (buf1, sem1) if b % 2 == 0 else (buf0, sem0)
            if b < 7:
                pltpu.async_copy(gref.at[sidx.at[par, b + 1]], obuf, osem)
            else:
                @pl.when(g + 1 < ng)
                def _():
                    pltpu.make_async_copy(idummy, sidx.at[nxt], semi).wait()
                    pltpu.make_async_copy(idummy, didx.at[nxt], semi).wait()
                    pltpu.async_copy(gref.at[sidx.at[nxt, 0]], obuf, osem)
            pltpu.make_async_copy(gdummy, buf, sem).wait()
            pltpu.sync_copy(buf, acc.at[didx.at[par, b]], add=True)
        return carry

    lax.fori_loop(0, ng, group, 0)
    plsc.subcore_barrier()

    @pl.when(s < 15)
    def _():
        pltpu.sync_copy(acc.at[pl.ds(s * _ZR, _ZR)],
                        out.at[c, pl.ds(s * _ZR, _ZR)])

    @pl.when(s == 15)
    def _():
        pltpu.sync_copy(acc.at[pl.ds(15 * _ZR, _N - 15 * _ZR)],
                        out.at[c, pl.ds(15 * _ZR, _N - 15 * _ZR)])


@functools.cache
def _sc_segsum(nchunks, split32):
    return pl.kernel(
        functools.partial(_sc_body, nchunks, split32),
        mesh=plsc.VectorSubcoreMesh(core_axis_name="c", subcore_axis_name="s"),
        out_type=jax.ShapeDtypeStruct((2, _N, 128), jnp.float32),
        scratch_types=[
            pltpu.VMEM((2, 8, _K), jnp.int32),
            pltpu.VMEM((2, 8, _K), jnp.int32),
            pltpu.VMEM((_K, 128), jnp.float32),
            pltpu.VMEM((_K, 128), jnp.float32),
            pltpu.VMEM_SHARED((_AR, 128), jnp.float32),
            pltpu.SemaphoreType.DMA,
            pltpu.SemaphoreType.DMA,
            pltpu.SemaphoreType.DMA,
        ],
    )


def _sc16(f0, src32, dst32, z128):
    return _sc_segsum(40, True)(f0, src32, dst32, z128)


def _sc128(p, src16, dst16, z128):
    return _sc_segsum(80, False)(p, src16, dst16, z128)


def _f0_body(x_ref, o_ref):
    xb = x_ref[...]
    r = xb.shape[0]
    li = lax.broadcasted_iota(jnp.int32, (r, 128), 1)
    v0 = xb[:, 0:1].astype(jnp.int32)
    v1 = xb[:, 1:2].astype(jnp.int32)
    oh0 = (li == v0).astype(jnp.float32)
    oh1 = ((li - 6) == v1).astype(jnp.float32)
    x2 = jnp.broadcast_to(xb[:, 2:3], (r, 128))
    x3 = jnp.broadcast_to(xb[:, 3:4], (r, 128))
    o_ref[...] = jnp.where(
        li < 6, oh0,
        jnp.where(li < 12, oh1,
                  jnp.where(li == 12, x2,
                            jnp.where(li == 13, x3,
                                      jnp.where(li == 14, 1.0, 0.0)))))


def _build_f0(x):
    return pl.pallas_call(
        _f0_body,
        grid=(_N // _R,),
        in_specs=[pl.BlockSpec((_R, 4), lambda i: (i, 0))],
        out_specs=pl.BlockSpec((_R, 128), lambda i: (i, 0)),
        out_shape=jax.ShapeDtypeStruct((_N, 128), jnp.float32),
    )(x)


def _write_sums(s_ref, h2):
    row = jnp.concatenate(
        [jnp.sum(h2, axis=0, keepdims=True),
         jnp.sum(h2 * h2, axis=0, keepdims=True)], axis=0)

    @pl.when(pl.program_id(0) == 0)
    def _():
        s_ref[...] = row

    @pl.when(pl.program_id(0) != 0)
    def _():
        s_ref[...] = s_ref[...] + row


def _conv0_body(f0_ref, g0_ref, emb_ref, w1t_ref, b1_ref, w2t_ref, b2_ref,
                eps_ref, p_ref, s_ref):
    f0b = f0_ref[:, 0:16]
    g0 = g0_ref[0, :, 0:16] + g0_ref[1, :, 0:16]
    z = (1.0 + eps_ref[0, 0]) * f0b + g0
    emb = emb_ref[...]
    z0a = jnp.dot(z[:, 0:6], emb, precision=_HI)
    z0b = jnp.dot(z[:, 6:12], emb, precision=_HI)
    pre = (jnp.dot(z0a, w1t_ref[0:128, :])
           + jnp.dot(z0b, w1t_ref[128:256, :])
           + jnp.dot(z[:, 12:14], w1t_ref[256:258, :])
           + b1_ref[...])
    h1 = jnp.maximum(pre, 0.0)
    h2 = jnp.maximum(jnp.dot(h1, w2t_ref[...]) + b2_ref[...], 0.0)
    p_ref[0] = h2[:, 0:128]
    p_ref[1] = h2[:, 128:256]
    _write_sums(s_ref, h2)


def _conv0(f0, g0p, emb, w1t, b1, w2t, b2, eps):
    return pl.pallas_call(
        _conv0_body,
        grid=(_N // _R,),
        in_specs=[
            pl.BlockSpec((_R, 128), lambda i: (i, 0)),
            pl.BlockSpec((2, _R, 128), lambda i: (0, i, 0)),
            pl.BlockSpec((6, 128), lambda i: (0, 0)),
            pl.BlockSpec((258, 256), lambda i: (0, 0)),
            pl.BlockSpec((1, 256), lambda i: (0, 0)),
            pl.BlockSpec((256, 256), lambda i: (0, 0)),
            pl.BlockSpec((1, 256), lambda i: (0, 0)),
            pl.BlockSpec((1, 1), lambda i: (0, 0)),
        ],
        out_specs=[
            pl.BlockSpec((2, _R, 128), lambda i: (0, i, 0)),
            pl.BlockSpec((2, 256), lambda i: (0, 0)),
        ],
        out_shape=[
            jax.ShapeDtypeStruct((2, _N, 128), jnp.float32),
            jax.ShapeDtypeStruct((2, 256), jnp.float32),
        ],
    )(f0, g0p, emb, w1t, b1, w2t, b2, eps)


def _bn_scale(sums, gamma, beta):
    mean = sums[0:1, :] * (1.0 / _N)
    var = sums[1:2, :] * (1.0 / _N) - mean * mean
    s = gamma * lax.rsqrt(var + 1e-5)
    t = beta - mean * s
    return s, t


def _convl_body(pp_ref, ag_ref, g0_ref, sp_ref, gam_ref, bet_ref, eps_ref,
                w1t_ref, b1_ref, w2t_ref, b2_ref, p_ref, s_ref):
    s_bn, t_bn = _bn_scale(sp_ref[...], gam_ref[...], bet_ref[...])
    pprev = jnp.concatenate([pp_ref[0], pp_ref[1]], axis=1)
    agg = jnp.concatenate([ag_ref[0], ag_ref[1]], axis=1)
    indeg = g0_ref[0, :, 14:15] + g0_ref[1, :, 14:15]
    e1 = 1.0 + eps_ref[0, 0]
    z = s_bn * (e1 * pprev + agg) + t_bn * (e1 + indeg)
    h1 = jnp.maximum(jnp.dot(z, w1t_ref[...]) + b1_ref[...], 0.0)
    h2 = jnp.maximum(jnp.dot(h1, w2t_ref[...]) + b2_ref[...], 0.0)
    p_ref[0] = h2[:, 0:128]
    p_ref[1] = h2[:, 128:256]
    _write_sums(s_ref, h2)


def _convl(pp, ag, g0p, sums_p, gam, bet, eps, w1t, b1, w2t, b2):
    return pl.pallas_call(
        _convl_body,
        grid=(_N // _R,),
        in_specs=[
            pl.BlockSpec((2, _R, 128), lambda i: (0, i, 0)),
            pl.BlockSpec((2, _R, 128), lambda i: (0, i, 0)),
            pl.BlockSpec((2, _R, 128), lambda i: (0, i, 0)),
            pl.BlockSpec((2, 256), lambda i: (0, 0)),
            pl.BlockSpec((1, 256), lambda i: (0, 0)),
            pl.BlockSpec((1, 256), lambda i: (0, 0)),
            pl.BlockSpec((1, 1), lambda i: (0, 0)),
            pl.BlockSpec((256, 256), lambda i: (0, 0)),
            pl.BlockSpec((1, 256), lambda i: (0, 0)),
            pl.BlockSpec((256, 256), lambda i: (0, 0)),
            pl.BlockSpec((1, 256), lambda i: (0, 0)),
        ],
        out_specs=[
            pl.BlockSpec((2, _R, 128), lambda i: (0, i, 0)),
            pl.BlockSpec((2, 256), lambda i: (0, 0)),
        ],
        out_shape=[
            jax.ShapeDtypeStruct((2, _N, 128), jnp.float32),
            jax.ShapeDtypeStruct((2, 256), jnp.float32),
        ],
    )(pp, ag, g0p, sums_p, gam, bet, eps, w1t, b1, w2t, b2)


def _head_body(p0_ref, p1_ref, p2_ref, s0_ref, s1_ref, s2_ref,
               gam_ref, bet_ref, bc_ref, br_ref,
               wiht_ref, bih_ref, whht_ref, bhh_ref,
               f1t_ref, f1b_ref, f2t_ref, f2b_ref, f3t_ref, f3b_ref,
               o_ref, x_scr):
    nch = _N // _R
    p_refs = (p0_ref, p1_ref, p2_ref)
    s_refs = (s0_ref, s1_ref, s2_ref)

    def build_x(k, carry):
        xc = jnp.full((_R, _H), -jnp.inf, jnp.float32)
        for l in range(3):
            s_bn, t_bn = _bn_scale(s_refs[l][...], gam_ref[l:l + 1, :],
                                   bet_ref[l:l + 1, :])
            pb = jnp.concatenate(
                [p_refs[l][0, pl.ds(k * _R, _R), :],
                 p_refs[l][1, pl.ds(k * _R, _R), :]], axis=1)
            xc = jnp.maximum(xc, s_bn * pb + t_bn)
        x_scr[pl.ds(k * _R, _R), :] = xc
        return carry

    lax.fori_loop(0, nch, build_x, 0)

    h = jnp.zeros((_G, _H), jnp.float32)
    cell = jnp.zeros((_G, _H), jnp.float32)
    qs = jnp.zeros((_G, 2 * _H), jnp.float32)
    for _step in range(3):
        gates = (jnp.dot(qs, wiht_ref[...]) + bih_ref[...]
                 + jnp.dot(h, whht_ref[...]) + bhh_ref[...])
        gi = jax.nn.sigmoid(gates[:, 0:256])
        gf = jax.nn.sigmoid(gates[:, 256:512])
        gg = jnp.tanh(gates[:, 512:768])
        go = jax.nn.sigmoid(gates[:, 768:1024])
        cell = gf * cell + gi * gg
        h = go * jnp.tanh(cell)

        def pass1(k, emax):
            xb = x_scr[pl.ds(k * _R, _R), :]
            bc = bc_ref[pl.ds(k * _R, _R), :].astype(jnp.int32)
            mk = bc == lax.broadcasted_iota(jnp.int32, (_R, _G), 1)
            qb = jnp.dot(mk.astype(jnp.float32), h, precision=_HI)
            e = jnp.sum(xb * qb, axis=1, keepdims=True)
            em = jnp.where(mk, e, -1e30)
            return jnp.maximum(emax, jnp.max(em, axis=0, keepdims=True))

        emax = lax.fori_loop(0, nch, pass1,
                             jnp.full((1, _G), -1e30, jnp.float32))

        def pass2(k, carry):
            den, run = carry
            xb = x_scr[pl.ds(k * _R, _R), :]
            bc = bc_ref[pl.ds(k * _R, _R), :].astype(jnp.int32)
            mk = bc == lax.broadcasted_iota(jnp.int32, (_R, _G), 1)
            qb = jnp.dot(mk.astype(jnp.float32), h, precision=_HI)
            e = jnp.sum(xb * qb, axis=1, keepdims=True)
            emaxb = jnp.max(jnp.where(mk, emax, -1e30), axis=1, keepdims=True)
            ex = jnp.exp(e - emaxb)
            br = br_ref[k].astype(jnp.int32)
            mkt = (br == lax.broadcasted_iota(jnp.int32, (_G, _R), 0)
                   ).astype(jnp.float32)
            den = den + jnp.dot(mkt, ex, precision=_HI)
            run = run + jnp.dot(mkt, ex * xb, precision=_HI)
            return den, run

        den, run = lax.fori_loop(
            0, nch, pass2,
            (jnp.zeros((_G, 1), jnp.float32), jnp.zeros((_G, _H), jnp.float32)))
        r = run / jnp.maximum(den, 1e-30)
        qs = jnp.concatenate([h, r], axis=1)

    h4 = jnp.maximum(jnp.dot(qs, f1t_ref[...]) + f1b_ref[...], 0.0)
    h5 = jnp.maximum(jnp.dot(h4, f2t_ref[...]) + f2b_ref[...], 0.0)
    o_ref[...] = jnp.dot(h5, f3t_ref[...]) + f3b_ref[...]


def _head(p0, p1, p2, s0, s1, s2, gam, bet, bc, br3, wiht, bih, whht, bhh,
          f1t, f1b, f2t, f2b, f3t, f3b):
    full = lambda shape: pl.BlockSpec(shape, lambda: tuple(0 for _ in shape))
    return pl.pallas_call(
        _head_body,
        grid=(),
        in_specs=[
            full((2, _N, 128)), full((2, _N, 128)), full((2, _N, 128)),
            full((2, 256)), full((2, 256)), full((2, 256)),
            full((3, 256)), full((3, 256)),
            full((_N, 1)), full((_N // _R, 1, _R)),
            full((2 * _H, 4 * _H)), full((1, 4 * _H)),
            full((_H, 4 * _H)), full((1, 4 * _H)),
            full((2 * _H, _H)), full((1, _H)),
            full((_H, _H // 2)), full((1, _H // 2)),
            full((_H // 2, 2)), full((1, 2)),
        ],
        out_specs=full((_G, 2)),
        out_shape=jax.ShapeDtypeStruct((_G, 2), jnp.float32),
        scratch_shapes=[pltpu.VMEM((_N, _H), jnp.float32)],
    )(p0, p1, p2, s0, s1, s2, gam, bet, bc, br3, wiht, bih, whht, bhh,
      f1t, f1b, f2t, f2b, f3t, f3b)


def kernel(x, params, edge_index, batch):
    src = edge_index[0].astype(jnp.int32)
    dst = edge_index[1].astype(jnp.int32)
    pad = _EPAD - _E
    srcp = jnp.concatenate([src, jnp.zeros((pad,), jnp.int32)])
    trash = _N + 8 + (jnp.arange(pad, dtype=jnp.int32) & 7)
    dstp = jnp.concatenate([dst, trash])
    src32 = srcp.reshape(32, 40, _K)
    dst32 = dstp.reshape(32, 40, _K)
    src16 = srcp.reshape(16, 80, _K)
    dst16 = dstp.reshape(16, 80, _K)
    z128 = jnp.zeros((_ZR, 128), jnp.float32)

    convs = params["convs"]
    w1t = [c["W1"].T for c in convs]
    w2t = [c["W2"].T for c in convs]
    b1 = [c["b1"].reshape(1, _H) for c in convs]
    b2 = [c["b2"].reshape(1, _H) for c in convs]
    eps = [c["eps"].reshape(1, 1) for c in convs]
    gam = jnp.stack([c["gamma"] for c in convs])
    bet = jnp.stack([c["beta"] for c in convs])

    f0 = _build_f0(x)
    g0p = _sc16(f0, src32, dst32, z128)
    p0, s0 = _conv0(f0, g0p, params["emb"], w1t[0], b1[0], w2t[0], b2[0],
                    eps[0])
    agg0 = _sc128(p0, src16, dst16, z128)
    p1, s1 = _convl(p0, agg0, g0p, s0, gam[0:1], bet[0:1], eps[1],
                    w1t[1], b1[1], w2t[1], b2[1])
    agg1 = _sc128(p1, src16, dst16, z128)
    p2, s2 = _convl(p1, agg1, g0p, s1, gam[1:2], bet[1:2], eps[2],
                    w1t[2], b1[2], w2t[2], b2[2])

    bc = batch.astype(jnp.float32).reshape(_N, 1)
    br3 = batch.astype(jnp.float32).reshape(_N // _R, 1, _R)
    logits = _head(
        p0, p1, p2, s0, s1, s2, gam, bet, bc, br3,
        params["Wih"].T, params["bih"].reshape(1, 4 * _H),
        params["Whh"].T, params["bhh"].reshape(1, 4 * _H),
        params["fc1W"].T, params["fc1b"].reshape(1, _H),
        params["fc2W"].T, params["fc2b"].reshape(1, _H // 2),
        params["fc3W"].T, params["fc3b"].reshape(1, 2))
    return logits
